# baseline (speedup 1.0000x reference)
import jax
import jax.numpy as jnp
from jax import lax
from jax.experimental import pallas as pl
from jax.experimental.pallas import tpu as pltpu
from jax.experimental.pallas import tpu_sc as plsc

N_NODES = 62
FEAT = 64
NPAD = 64
N_TRIL = N_NODES * (N_NODES + 1) // 2
P_PAD_LEN = 2048
HP = jax.lax.Precision.HIGHEST


def _sc_build_a_body(p_hbm, a_hbm, p_v, row_v):
    wid = lax.axis_index("s") * 2 + lax.axis_index("c")
    pltpu.sync_copy(p_hbm, p_v)
    for r in range(2):
        row = wid * 2 + r
        row_b = jnp.full((16,), row, jnp.int32)
        for j in range(NPAD // 16):
            cvec = lax.iota(jnp.int32, 16) + (j * 16)
            hi = jnp.maximum(cvec, row_b)
            lo = jnp.minimum(cvec, row_b)
            idx = lax.shift_right_logical(hi * (hi + 1), 1) + lo
            valid = (cvec != row_b) & (cvec < N_NODES) & (row_b < N_NODES)
            iv = jnp.where(valid, idx, P_PAD_LEN - 1)
            row_v[pl.ds(j * 16, 16)] = plsc.load_gather(p_v, [iv])
        pltpu.sync_copy(row_v, a_hbm.at[row])


def _make_sc_build_a():
    return pl.kernel(
        _sc_build_a_body,
        out_type=jax.ShapeDtypeStruct((NPAD, NPAD), jnp.float32),
        mesh=plsc.VectorSubcoreMesh(
            core_axis_name="c", subcore_axis_name="s",
            num_cores=2, num_subcores=16),
        scratch_types=[
            pltpu.VMEM((P_PAD_LEN,), jnp.float32),
            pltpu.VMEM((NPAD,), jnp.float32),
        ],
        compiler_params=pltpu.CompilerParams(needs_layout_passes=False),
    )


N_DMA = 4


def _eeg_kernel(xr_ref, a_ref, w0a_ref, w1a_ref, w0b_ref, w1b_ref,
                ba_ref, bb_ref, fcw_ref, fcb_ref, out_ref, xv_ref, sems):
    rows = 256 // N_DMA
    copies = [
        pltpu.make_async_copy(
            xr_ref.at[pl.ds(i * rows, rows), :],
            xv_ref.at[pl.ds(i * rows, rows), :],
            sems.at[i])
        for i in range(N_DMA)
    ]
    for c in copies:
        c.start()

    A = a_ref[:, :]
    deg_c = jnp.sum(A, axis=1, keepdims=True)
    deg_r = jnp.sum(A, axis=0, keepdims=True)
    dis_c = jnp.where(deg_c > 0, 1.0 / jnp.sqrt(jnp.maximum(deg_c, 1e-12)), 0.0)
    dis_r = jnp.where(deg_r > 0, 1.0 / jnp.sqrt(jnp.maximum(deg_r, 1e-12)), 0.0)
    S = -(dis_c * A * dis_r)
    srow = jnp.sum(S, axis=0, keepdims=True)

    w0a = w0a_ref[:, :]
    w1a = w1a_ref[:, :]
    w0b = w0b_ref[:, :]
    w1b = w1b_ref[:, :]
    m0 = jnp.dot(w0a, w0b, precision=HP)
    m1 = jnp.dot(w0a, w1b, precision=HP) + jnp.dot(w1a, w0b, precision=HP)
    m2 = jnp.dot(w1a, w1b, precision=HP)
    alpha = jnp.dot(ba_ref[:, :], w0b, precision=HP) + bb_ref[:, :]
    beta = jnp.dot(ba_ref[:, :], w1b, precision=HP)

    R0 = jnp.concatenate(
        [jnp.transpose(fcw_ref[:, :]),
         jnp.zeros((NPAD - N_NODES, 3), jnp.float32)], axis=0)
    R1 = jnp.dot(S, R0, precision=HP)
    R2 = jnp.dot(S, R1, precision=HP)

    q = (m0.reshape(1, FEAT, 1) * jax.lax.slice(R0, (0, 0), (N_NODES, 3)).reshape(N_NODES, 1, 3)
         + m1.reshape(1, FEAT, 1) * jax.lax.slice(R1, (0, 0), (N_NODES, 3)).reshape(N_NODES, 1, 3)
         + m2.reshape(1, FEAT, 1) * jax.lax.slice(R2, (0, 0), (N_NODES, 3)).reshape(N_NODES, 1, 3))
    Q = q.reshape(N_NODES * FEAT, 3)

    bias = jnp.dot(alpha + beta * srow, R0, precision=HP)
    for c in copies:
        c.wait()
    out_ref[:, :] = (jnp.dot(xv_ref[:, :], Q, precision=HP)
                     + bias + fcb_ref[:, :])


def kernel(x, edge_index, y, batch, edge_weight_param, W0a, W1a, ba,
           W0b, W1b, bb, fcW, fcb):
    bsz = y.shape[0]
    xr = x.reshape(bsz, N_NODES * FEAT)
    p_pad = jnp.zeros((P_PAD_LEN,), jnp.float32).at[:N_TRIL].set(edge_weight_param)
    ba_r = ba.reshape(1, FEAT)
    bb_r = bb.reshape(1, 1)
    fcb_r = fcb.reshape(1, 3)

    a = _make_sc_build_a()(p_pad)

    vmem = lambda: pl.BlockSpec(memory_space=pltpu.VMEM)
    return pl.pallas_call(
        _eeg_kernel,
        in_specs=[pl.BlockSpec(memory_space=pl.ANY),
                  vmem(), vmem(), vmem(), vmem(), vmem(),
                  vmem(), vmem(), vmem(), vmem()],
        out_shape=jax.ShapeDtypeStruct((bsz, 3), jnp.float32),
        scratch_shapes=[
            pltpu.VMEM((bsz, N_NODES * FEAT), jnp.float32),
            pltpu.SemaphoreType.DMA((N_DMA,)),
        ],
    )(xr, a, W0a, W1a, W0b, W1b, ba_r, bb_r, fcW, fcb_r)


# speedup vs baseline: 1.0067x; 1.0067x over previous
import jax
import jax.numpy as jnp
from jax import lax
from jax.experimental import pallas as pl
from jax.experimental.pallas import tpu as pltpu
from jax.experimental.pallas import tpu_sc as plsc

N_NODES = 62
FEAT = 64
NPAD = 64
N_TRIL = N_NODES * (N_NODES + 1) // 2
P_PAD_LEN = 2048
HP = jax.lax.Precision.HIGHEST


def _sc_build_a_body(p_hbm, a_hbm, p_v, row_v):
    wid = lax.axis_index("s") * 2 + lax.axis_index("c")
    pltpu.sync_copy(p_hbm, p_v)
    for r in range(2):
        row = wid * 2 + r
        row_b = jnp.full((16,), row, jnp.int32)
        for j in range(NPAD // 16):
            cvec = lax.iota(jnp.int32, 16) + (j * 16)
            hi = jnp.maximum(cvec, row_b)
            lo = jnp.minimum(cvec, row_b)
            idx = lax.shift_right_logical(hi * (hi + 1), 1) + lo
            valid = (cvec != row_b) & (cvec < N_NODES) & (row_b < N_NODES)
            iv = jnp.where(valid, idx, P_PAD_LEN - 1)
            row_v[pl.ds(j * 16, 16)] = plsc.load_gather(p_v, [iv])
        pltpu.sync_copy(row_v, a_hbm.at[row])


def _make_sc_build_a():
    return pl.kernel(
        _sc_build_a_body,
        out_type=jax.ShapeDtypeStruct((NPAD, NPAD), jnp.float32),
        mesh=plsc.VectorSubcoreMesh(
            core_axis_name="c", subcore_axis_name="s",
            num_cores=2, num_subcores=16),
        scratch_types=[
            pltpu.VMEM((P_PAD_LEN,), jnp.float32),
            pltpu.VMEM((NPAD,), jnp.float32),
        ],
        compiler_params=pltpu.CompilerParams(needs_layout_passes=False),
    )


def _eeg_kernel(xr_ref, a_ref, w0a_ref, w1a_ref, w0b_ref, w1b_ref,
                ba_ref, bb_ref, fcw_ref, fcb_ref, out_ref):
    A = a_ref[:, :]
    deg_c = jnp.sum(A, axis=1, keepdims=True)
    deg_r = jnp.sum(A, axis=0, keepdims=True)
    dis_c = jnp.where(deg_c > 0, 1.0 / jnp.sqrt(jnp.maximum(deg_c, 1e-12)), 0.0)
    dis_r = jnp.where(deg_r > 0, 1.0 / jnp.sqrt(jnp.maximum(deg_r, 1e-12)), 0.0)
    S = -(dis_c * A * dis_r)
    srow = jnp.sum(S, axis=0, keepdims=True)

    w0a = w0a_ref[:, :]
    w1a = w1a_ref[:, :]
    w0b = w0b_ref[:, :]
    w1b = w1b_ref[:, :]
    m0 = jnp.dot(w0a, w0b, precision=HP)
    m1 = jnp.dot(w0a, w1b, precision=HP) + jnp.dot(w1a, w0b, precision=HP)
    m2 = jnp.dot(w1a, w1b, precision=HP)
    alpha = jnp.dot(ba_ref[:, :], w0b, precision=HP) + bb_ref[:, :]
    beta = jnp.dot(ba_ref[:, :], w1b, precision=HP)

    R0 = jnp.concatenate(
        [jnp.transpose(fcw_ref[:, :]),
         jnp.zeros((NPAD - N_NODES, 3), jnp.float32)], axis=0)
    R1 = jnp.dot(S, R0, precision=HP)
    R2 = jnp.dot(S, R1, precision=HP)

    q = (m0.reshape(1, FEAT, 1) * jax.lax.slice(R0, (0, 0), (N_NODES, 3)).reshape(N_NODES, 1, 3)
         + m1.reshape(1, FEAT, 1) * jax.lax.slice(R1, (0, 0), (N_NODES, 3)).reshape(N_NODES, 1, 3)
         + m2.reshape(1, FEAT, 1) * jax.lax.slice(R2, (0, 0), (N_NODES, 3)).reshape(N_NODES, 1, 3))
    Q = q.reshape(N_NODES * FEAT, 3)

    bias = jnp.dot(alpha + beta * srow, R0, precision=HP)
    out_ref[:, :] = (jnp.dot(xr_ref[:, :], Q, precision=HP)
                     + bias + fcb_ref[:, :])


def kernel(x, edge_index, y, batch, edge_weight_param, W0a, W1a, ba,
           W0b, W1b, bb, fcW, fcb):
    bsz = y.shape[0]
    xr = x.reshape(bsz, N_NODES * FEAT)
    p_pad = jnp.zeros((P_PAD_LEN,), jnp.float32).at[:N_TRIL].set(edge_weight_param)
    ba_r = ba.reshape(1, FEAT)
    bb_r = bb.reshape(1, 1)
    fcb_r = fcb.reshape(1, 3)

    a = _make_sc_build_a()(p_pad)

    return pl.pallas_call(
        _eeg_kernel,
        out_shape=jax.ShapeDtypeStruct((bsz, 3), jnp.float32),
    )(xr, a, W0a, W1a, W0b, W1b, ba_r, bb_r, fcW, fcb_r)


# speedup vs baseline: 1.0884x; 1.0811x over previous
import jax
import jax.numpy as jnp
from jax import lax
from jax.experimental import pallas as pl
from jax.experimental.pallas import tpu as pltpu
from jax.experimental.pallas import tpu_sc as plsc

N_NODES = 62
FEAT = 64
NPAD = 64
N_TRIL = N_NODES * (N_NODES + 1) // 2
P_PAD_LEN = 2048
HP = jax.lax.Precision.HIGHEST


def _sc_build_a_body(p_hbm, a_hbm, p_v, row_v):
    wid = lax.axis_index("s") * 2 + lax.axis_index("c")
    pltpu.sync_copy(p_hbm, p_v)
    for r in range(2):
        row = wid * 2 + r
        row_b = jnp.full((16,), row, jnp.int32)
        for j in range(NPAD // 16):
            cvec = lax.iota(jnp.int32, 16) + (j * 16)
            hi = jnp.maximum(cvec, row_b)
            lo = jnp.minimum(cvec, row_b)
            idx = lax.shift_right_logical(hi * (hi + 1), 1) + lo
            valid = (cvec != row_b) & (cvec < N_NODES) & (row_b < N_NODES)
            iv = jnp.where(valid, idx, P_PAD_LEN - 1)
            row_v[pl.ds(j * 16, 16)] = plsc.load_gather(p_v, [iv])
        pltpu.sync_copy(row_v, a_hbm.at[row])


def _make_sc_build_a():
    return pl.kernel(
        _sc_build_a_body,
        out_type=jax.ShapeDtypeStruct((NPAD, NPAD), jnp.float32),
        mesh=plsc.VectorSubcoreMesh(
            core_axis_name="c", subcore_axis_name="s",
            num_cores=2, num_subcores=16),
        scratch_types=[
            pltpu.VMEM((P_PAD_LEN,), jnp.float32),
            pltpu.VMEM((NPAD,), jnp.float32),
        ],
        compiler_params=pltpu.CompilerParams(needs_layout_passes=False),
    )


def _eeg_kernel(xr_ref, a_ref, w0a_ref, w1a_ref, w0b_ref, w1b_ref,
                ba_ref, bb_ref, fcw_ref, fcb_ref, out_ref):
    A = a_ref[:, :]
    deg_c = jnp.sum(A, axis=1, keepdims=True)
    deg_r = jnp.sum(A, axis=0, keepdims=True)
    dis_c = jnp.where(deg_c > 0, 1.0 / jnp.sqrt(jnp.maximum(deg_c, 1e-12)), 0.0)
    dis_r = jnp.where(deg_r > 0, 1.0 / jnp.sqrt(jnp.maximum(deg_r, 1e-12)), 0.0)
    S = -(dis_c * A * dis_r)
    srow = jnp.sum(S, axis=0, keepdims=True)

    w0a = w0a_ref[:, :]
    w1a = w1a_ref[:, :]
    w0b = w0b_ref[:, :]
    w1b = w1b_ref[:, :]
    m0 = jnp.dot(w0a, w0b, precision=HP)
    m1 = jnp.dot(w0a, w1b, precision=HP) + jnp.dot(w1a, w0b, precision=HP)
    m2 = jnp.dot(w1a, w1b, precision=HP)
    alpha = jnp.dot(ba_ref[:, :], w0b, precision=HP) + bb_ref[:, :]
    beta = jnp.dot(ba_ref[:, :], w1b, precision=HP)

    R0 = jnp.concatenate(
        [jnp.transpose(fcw_ref[:, :]),
         jnp.zeros((NPAD - N_NODES, 3), jnp.float32)], axis=0)
    R1 = jnp.dot(S, R0, precision=HP)
    R2 = jnp.dot(S, R1, precision=HP)

    q = (m0.reshape(1, FEAT, 1) * jax.lax.slice(R0, (0, 0), (N_NODES, 3)).reshape(N_NODES, 1, 3)
         + m1.reshape(1, FEAT, 1) * jax.lax.slice(R1, (0, 0), (N_NODES, 3)).reshape(N_NODES, 1, 3)
         + m2.reshape(1, FEAT, 1) * jax.lax.slice(R2, (0, 0), (N_NODES, 3)).reshape(N_NODES, 1, 3))
    Q = q.reshape(N_NODES * FEAT, 3)

    bias = jnp.dot(alpha + beta * srow, R0, precision=HP)
    out_ref[:, :] = (jnp.dot(xr_ref[:, :], Q)
                     + bias + fcb_ref[:, :])


def kernel(x, edge_index, y, batch, edge_weight_param, W0a, W1a, ba,
           W0b, W1b, bb, fcW, fcb):
    bsz = y.shape[0]
    xr = x.reshape(bsz, N_NODES * FEAT)
    p_pad = jnp.zeros((P_PAD_LEN,), jnp.float32).at[:N_TRIL].set(edge_weight_param)
    ba_r = ba.reshape(1, FEAT)
    bb_r = bb.reshape(1, 1)
    fcb_r = fcb.reshape(1, 3)

    a = _make_sc_build_a()(p_pad)

    return pl.pallas_call(
        _eeg_kernel,
        out_shape=jax.ShapeDtypeStruct((bsz, 3), jnp.float32),
    )(xr, a, W0a, W1a, W0b, W1b, ba_r, bb_r, fcW, fcb_r)
